# baseline (speedup 1.0000x reference)
import jax
import jax.numpy as jnp
from jax import lax
from jax.experimental import pallas as pl
from jax.experimental.pallas import tpu as pltpu
from jax.experimental.pallas import tpu_sc as plsc

_RADIUS2 = 0.2 * 0.2
_NSAMPLE = 64
_B, _N, _S = 8, 4096, 1024
_L = 16
_NCHUNK = _N // _L
_QPW = 256
_GPB = _S // _QPW
_U = 8
_BUF = 192


def _sc_body(p1x, p1y, p1z, q2x, q2y, q2z, idx_hbm, cnt_hbm,
             px, py, pz, qx, qy, qz, buf, oidx, ocnt):
    c = lax.axis_index("c")
    s = lax.axis_index("s")
    wid = s * 2 + c
    b = wid // _GPB
    q0 = (wid % _GPB) * _QPW

    pltpu.sync_copy(p1x.at[pl.ds(b * _N, _N)], px.at[pl.ds(0, _N)])
    pltpu.sync_copy(p1y.at[pl.ds(b * _N, _N)], py.at[pl.ds(0, _N)])
    pltpu.sync_copy(p1z.at[pl.ds(b * _N, _N)], pz.at[pl.ds(0, _N)])
    qbase = b * _S + q0
    pltpu.sync_copy(q2x.at[pl.ds(qbase, _QPW)], qx)
    pltpu.sync_copy(q2y.at[pl.ds(qbase, _QPW)], qy)
    pltpu.sync_copy(q2z.at[pl.ds(qbase, _QPW)], qz)

    lane = lax.iota(jnp.int32, _L)
    zeros16 = jnp.zeros((_L,), jnp.int32)

    def per_group(g, carry):
        qxg = qx[pl.ds(g * _L, _L)]
        qyg = qy[pl.ds(g * _L, _L)]
        qzg = qz[pl.ds(g * _L, _L)]
        for l in range(_L):
            i = g * _L + l
            qxv = jnp.full((_L,), qxg[l], jnp.float32)
            qyv = jnp.full((_L,), qyg[l], jnp.float32)
            qzv = jnp.full((_L,), qzg[l], jnp.float32)
            buf[pl.ds(0, _L)] = zeros16

            def load(j):
                out = []
                for u in range(_U):
                    base = (j + u) * _L
                    out.extend((px[pl.ds(base, _L)],
                                py[pl.ds(base, _L)],
                                pz[pl.ds(base, _L)]))
                return tuple(out)

            def cond(c):
                j, cnt = c[0], c[1]
                return jnp.logical_and(j < _NCHUNK, cnt < _NSAMPLE)

            def body(c, qxv=qxv, qyv=qyv, qzv=qzv):
                j, cnt = c[0], c[1]
                coords = c[2:]
                nxt = load(j + _U)
                for u in range(_U):
                    cx = coords[3 * u]
                    cy = coords[3 * u + 1]
                    cz = coords[3 * u + 2]
                    dx = qxv - cx
                    dy = qyv - cy
                    dz = qzv - cz
                    d2 = dx * dx + dy * dy + dz * dz
                    m = d2 < _RADIUS2
                    pc = plsc.all_reduce_population_count(m)[0]
                    plsc.store_compressed(buf.at[pl.ds(cnt, _L)],
                                          (j + u) * _L + lane, mask=m)
                    cnt = cnt + pc
                return (j + _U, cnt) + nxt

            fin = lax.while_loop(cond, body, (0, 0) + load(0))
            cnt = fin[1]
            cntc = jnp.minimum(cnt, _NSAMPLE)
            cntv = jnp.full((_L,), cntc, jnp.int32)
            fv = buf[pl.ds(0, _L)]
            first = jnp.full((_L,), fv[0], jnp.int32)
            for k in range(_NSAMPLE // _L):
                v = buf[pl.ds(k * _L, _L)]
                v = jnp.where(lane + (k * _L) < cntv, v, first)
                oidx[pl.ds(i * _NSAMPLE + k * _L, _L)] = v
            plsc.store_scatter(ocnt, [jnp.full((_L,), i, jnp.int32)], cntv,
                               mask=lane == 0)
        return carry

    lax.fori_loop(0, _QPW // _L, per_group, 0)

    pltpu.sync_copy(oidx, idx_hbm.at[pl.ds((b * _S + q0) * _NSAMPLE,
                                           _QPW * _NSAMPLE)])
    pltpu.sync_copy(ocnt, cnt_hbm.at[pl.ds(b * _S + q0, _QPW)])


@jax.jit
def kernel(xyz1, xyz2):
    p1 = jnp.transpose(xyz1, (2, 0, 1)).reshape(3, _B * _N)
    q2 = jnp.transpose(xyz2, (2, 0, 1)).reshape(3, _B * _S)
    run = pl.kernel(
        _sc_body,
        out_type=(
            jax.ShapeDtypeStruct((_B * _S * _NSAMPLE,), jnp.int32),
            jax.ShapeDtypeStruct((_B * _S,), jnp.int32),
        ),
        mesh=plsc.VectorSubcoreMesh(core_axis_name="c", subcore_axis_name="s"),
        compiler_params=pltpu.CompilerParams(needs_layout_passes=False),
        scratch_types=[
            pltpu.VMEM((_N + _U * _L,), jnp.float32),
            pltpu.VMEM((_N + _U * _L,), jnp.float32),
            pltpu.VMEM((_N + _U * _L,), jnp.float32),
            pltpu.VMEM((_QPW,), jnp.float32),
            pltpu.VMEM((_QPW,), jnp.float32),
            pltpu.VMEM((_QPW,), jnp.float32),
            pltpu.VMEM((_BUF,), jnp.int32),
            pltpu.VMEM((_QPW * _NSAMPLE,), jnp.int32),
            pltpu.VMEM((_QPW,), jnp.int32),
        ],
    )
    idx, cnt = run(p1[0], p1[1], p1[2], q2[0], q2[1], q2[2])
    return idx.reshape(_B, _S, _NSAMPLE), cnt.reshape(_B, _S)


# speedup vs baseline: 1.1373x; 1.1373x over previous
import jax
import jax.numpy as jnp
from jax import lax
from jax.experimental import pallas as pl
from jax.experimental.pallas import tpu as pltpu
from jax.experimental.pallas import tpu_sc as plsc

_RADIUS2 = 0.2 * 0.2
_NSAMPLE = 64
_B, _N, _S = 8, 4096, 1024
_L = 16
_NCHUNK = _N // _L
_QPW = 256
_GPB = _S // _QPW
_U = 16
_BUF = 320


def _sc_body(p1x, p1y, p1z, q2x, q2y, q2z, idx_hbm, cnt_hbm,
             px, py, pz, qx, qy, qz, buf, oidx, ocnt):
    c = lax.axis_index("c")
    s = lax.axis_index("s")
    wid = s * 2 + c
    b = wid // _GPB
    q0 = (wid % _GPB) * _QPW

    pltpu.sync_copy(p1x.at[pl.ds(b * _N, _N)], px.at[pl.ds(0, _N)])
    pltpu.sync_copy(p1y.at[pl.ds(b * _N, _N)], py.at[pl.ds(0, _N)])
    pltpu.sync_copy(p1z.at[pl.ds(b * _N, _N)], pz.at[pl.ds(0, _N)])
    qbase = b * _S + q0
    pltpu.sync_copy(q2x.at[pl.ds(qbase, _QPW)], qx)
    pltpu.sync_copy(q2y.at[pl.ds(qbase, _QPW)], qy)
    pltpu.sync_copy(q2z.at[pl.ds(qbase, _QPW)], qz)

    lane = lax.iota(jnp.int32, _L)
    zeros16 = jnp.zeros((_L,), jnp.int32)

    def per_group(g, carry):
        qxg = qx[pl.ds(g * _L, _L)]
        qyg = qy[pl.ds(g * _L, _L)]
        qzg = qz[pl.ds(g * _L, _L)]
        for l in range(_L):
            i = g * _L + l
            qxv = jnp.full((_L,), qxg[l], jnp.float32)
            qyv = jnp.full((_L,), qyg[l], jnp.float32)
            qzv = jnp.full((_L,), qzg[l], jnp.float32)
            buf[pl.ds(0, _L)] = zeros16

            def load(j):
                out = []
                for u in range(_U):
                    base = (j + u) * _L
                    out.extend((px[pl.ds(base, _L)],
                                py[pl.ds(base, _L)],
                                pz[pl.ds(base, _L)]))
                return tuple(out)

            def cond(c):
                j, cnt = c[0], c[1]
                return jnp.logical_and(j < _NCHUNK, cnt < _NSAMPLE)

            def body(c, qxv=qxv, qyv=qyv, qzv=qzv):
                j, cnt = c[0], c[1]
                coords = c[2:]
                nxt = load(j + _U)
                for u in range(_U):
                    cx = coords[3 * u]
                    cy = coords[3 * u + 1]
                    cz = coords[3 * u + 2]
                    dx = qxv - cx
                    dy = qyv - cy
                    dz = qzv - cz
                    d2 = dx * dx + dy * dy + dz * dz
                    m = d2 < _RADIUS2
                    pc = plsc.all_reduce_population_count(m)[0]
                    plsc.store_compressed(buf.at[pl.ds(cnt, _L)],
                                          (j + u) * _L + lane, mask=m)
                    cnt = cnt + pc
                return (j + _U, cnt) + nxt

            fin = lax.while_loop(cond, body, (0, 0) + load(0))
            cnt = fin[1]
            cntc = jnp.minimum(cnt, _NSAMPLE)
            cntv = jnp.full((_L,), cntc, jnp.int32)
            fv = buf[pl.ds(0, _L)]
            first = jnp.full((_L,), fv[0], jnp.int32)
            for k in range(_NSAMPLE // _L):
                v = buf[pl.ds(k * _L, _L)]
                v = jnp.where(lane + (k * _L) < cntv, v, first)
                oidx[pl.ds(i * _NSAMPLE + k * _L, _L)] = v
            plsc.store_scatter(ocnt, [jnp.full((_L,), i, jnp.int32)], cntv,
                               mask=lane == 0)
        return carry

    lax.fori_loop(0, _QPW // _L, per_group, 0)

    pltpu.sync_copy(oidx, idx_hbm.at[pl.ds((b * _S + q0) * _NSAMPLE,
                                           _QPW * _NSAMPLE)])
    pltpu.sync_copy(ocnt, cnt_hbm.at[pl.ds(b * _S + q0, _QPW)])


@jax.jit
def kernel(xyz1, xyz2):
    p1 = jnp.transpose(xyz1, (2, 0, 1)).reshape(3, _B * _N)
    q2 = jnp.transpose(xyz2, (2, 0, 1)).reshape(3, _B * _S)
    run = pl.kernel(
        _sc_body,
        out_type=(
            jax.ShapeDtypeStruct((_B * _S * _NSAMPLE,), jnp.int32),
            jax.ShapeDtypeStruct((_B * _S,), jnp.int32),
        ),
        mesh=plsc.VectorSubcoreMesh(core_axis_name="c", subcore_axis_name="s"),
        compiler_params=pltpu.CompilerParams(needs_layout_passes=False),
        scratch_types=[
            pltpu.VMEM((_N + _U * _L,), jnp.float32),
            pltpu.VMEM((_N + _U * _L,), jnp.float32),
            pltpu.VMEM((_N + _U * _L,), jnp.float32),
            pltpu.VMEM((_QPW,), jnp.float32),
            pltpu.VMEM((_QPW,), jnp.float32),
            pltpu.VMEM((_QPW,), jnp.float32),
            pltpu.VMEM((_BUF,), jnp.int32),
            pltpu.VMEM((_QPW * _NSAMPLE,), jnp.int32),
            pltpu.VMEM((_QPW,), jnp.int32),
        ],
    )
    idx, cnt = run(p1[0], p1[1], p1[2], q2[0], q2[1], q2[2])
    return idx.reshape(_B, _S, _NSAMPLE), cnt.reshape(_B, _S)
